# baseline (speedup 1.0000x reference)
import functools

import jax
import jax.numpy as jnp
from jax import lax
from jax.experimental import pallas as pl
from jax.experimental.pallas import tpu as pltpu
from jax.experimental.pallas import tpu_sc as plsc

BATCH = 1024
SEQ = 50
HID = 64
NITEM = 100000
SEQ_PAD = 64
NWORK = 32
ROWS_PER_W = BATCH // NWORK
BB = 256
NSEG = NITEM // 16
RING = 4


def _memset_body(o_ref):
    o_ref[...] = jnp.zeros_like(o_ref)


def _make_zeros():
    return pl.pallas_call(
        _memset_body,
        grid=(NWORK,),
        out_specs=pl.BlockSpec((BATCH // NWORK, NITEM), lambda i: (i, 0)),
        out_shape=jax.ShapeDtypeStruct((BATCH, NITEM), jnp.float32),
    )()


def _probs_body(am_ref, lm_ref, item_ref, wr_ref, ur_ref, vrw_ref,
                idx_out, val_out):
    am = am_ref[...]
    lm = lm_ref[...]
    item = item_ref[...]
    wr = wr_ref[...]
    ur = ur_ref[...]
    vrw = vrw_ref[...]

    amu = lax.dot_general(am, ur, (((2,), (1,)), ((), ())),
                          preferred_element_type=jnp.float32)
    lmw = lax.dot_general(lm, wr, (((1,), (1,)), ((), ())),
                          preferred_element_type=jnp.float32)
    t = jnp.tanh(amu + lmw[:, None, :])
    s = jnp.sum(t * vrw[0][None, None, :], axis=-1)
    s = s - jnp.max(s, axis=-1, keepdims=True)
    e = jnp.exp(s)
    p = e / jnp.sum(e, axis=-1, keepdims=True)

    it = item[:, :SEQ]
    eq = it[:, :, None] == it[:, None, :]
    comb = jnp.sum(jnp.where(eq, p[:, None, :], 0.0), axis=-1)
    qlt = (jnp.arange(SEQ)[:, None] > jnp.arange(SEQ)[None, :])[None]
    firsti = jnp.where(
        jnp.sum(jnp.where(eq & qlt, 1, 0), axis=-1) == 0, 1, 0)

    lane = (jnp.arange(SEQ_PAD, dtype=jnp.int32) % 16)[None, :]
    pad_cols = SEQ_PAD - SEQ
    first_p = jnp.pad(firsti, ((0, 0), (0, pad_cols))) > 0
    comb_p = jnp.pad(comb, ((0, 0), (0, pad_cols)))
    it_p = jnp.pad(it, ((0, 0), (0, pad_cols)))
    idx_out[...] = jnp.where(first_p, it_p, NITEM + lane).astype(jnp.int32)
    val_out[...] = jnp.where(first_p, comb_p, 0.0)


def _compute_scatter_args(all_memory, last_memory, seq_item, Wr, Ur, Vr_w):
    grid = BATCH // BB
    return pl.pallas_call(
        _probs_body,
        grid=(grid,),
        in_specs=[
            pl.BlockSpec((BB, SEQ, HID), lambda i: (i, 0, 0)),
            pl.BlockSpec((BB, HID), lambda i: (i, 0)),
            pl.BlockSpec((BB, SEQ_PAD), lambda i: (i, 0)),
            pl.BlockSpec((HID, HID), lambda i: (0, 0)),
            pl.BlockSpec((HID, HID), lambda i: (0, 0)),
            pl.BlockSpec((1, HID), lambda i: (0, 0)),
        ],
        out_specs=[
            pl.BlockSpec((BB, SEQ_PAD), lambda i: (i, 0)),
            pl.BlockSpec((BB, SEQ_PAD), lambda i: (i, 0)),
        ],
        out_shape=[
            jax.ShapeDtypeStruct((BATCH, SEQ_PAD), jnp.int32),
            jax.ShapeDtypeStruct((BATCH, SEQ_PAD), jnp.float32),
        ],
    )(all_memory, last_memory, seq_item, Wr, Ur, Vr_w)


@functools.cache
def _make_scatter_kernel():
    return pl.kernel(
        _scatter_body,
        out_type=(),
        mesh=plsc.VectorSubcoreMesh(core_axis_name="c", subcore_axis_name="s",
                                    num_cores=2, num_subcores=16),
        compiler_params=pltpu.CompilerParams(needs_layout_passes=False,
                                             use_tc_tiling_on_sc=False),
        scratch_types=[
            pltpu.VMEM((NITEM + 16,), jnp.float32),
            pltpu.VMEM((ROWS_PER_W * SEQ_PAD,), jnp.int32),
            pltpu.VMEM((ROWS_PER_W * SEQ_PAD,), jnp.float32),
            [pltpu.VMEM((SEQ_PAD,), jnp.int32) for _ in range(RING)],
            [pltpu.VMEM((SEQ_PAD, 16), jnp.float32) for _ in range(RING)],
            pltpu.SemaphoreType.DMA,
        ],
    )


def _scatter_body(idx_hbm, val_hbm, out3, row_buf, idx_v, val_v,
                  sidx, sdat, sem):
    wid = lax.axis_index("s") * 2 + lax.axis_index("c")
    base = wid * ROWS_PER_W

    pltpu.sync_copy(idx_hbm.at[pl.ds(base * SEQ_PAD, ROWS_PER_W * SEQ_PAD)],
                    idx_v)
    pltpu.sync_copy(val_hbm.at[pl.ds(base * SEQ_PAD, ROWS_PER_W * SEQ_PAD)],
                    val_v)

    zeros16 = jnp.zeros((16,), jnp.float32)

    def zinit(i, carry):
        row_buf[pl.ds(i * 16, 16)] = zeros16
        return carry

    lax.fori_loop(0, (NITEM + 16) // 16, zinit, 0)

    def one_row(r, slot):
        off = r * SEQ_PAD
        for k in range(SEQ_PAD // 16):
            idx = idx_v[pl.ds(off + k * 16, 16)]
            val = val_v[pl.ds(off + k * 16, 16)]
            plsc.addupdate_scatter(row_buf, [idx], val)
        row_gseg = (base + r) * NSEG
        vec0 = idx_v[pl.ds(off, 16)]
        seg0 = (vec0[0] >> 4) << 4
        for k in range(SEQ_PAD // 16):
            vec = idx_v[pl.ds(off + k * 16, 16)]
            startv = jnp.where(vec < NITEM, (vec >> 4) << 4, seg0)
            sidx[slot][pl.ds(k * 16, 16)] = row_gseg + (startv >> 4)
            for jj in range(16):
                sdat[slot][k * 16 + jj, :] = row_buf[pl.ds(startv[jj], 16)]
        pltpu.async_copy(sdat[slot], out3.at[sidx[slot]], sem)
        for k in range(SEQ_PAD // 16):
            idx = idx_v[pl.ds(off + k * 16, 16)]
            plsc.store_scatter(row_buf, [idx], zeros16)

    for s in range(RING):
        one_row(s, s)

    def group(g, carry):
        for s in range(RING):
            pltpu.make_async_copy(sdat[s], out3.at[sidx[s]], sem).wait()
            one_row(g * RING + s, s)
        return carry

    lax.fori_loop(1, ROWS_PER_W // RING, group, 0)

    for s in range(RING):
        pltpu.make_async_copy(sdat[s], out3.at[sidx[s]], sem).wait()


def kernel(all_memory, last_memory, seq_item, Wr, Ur, Vr_w, Vr_b):
    del Vr_b
    seq_item = seq_item.astype(jnp.int32)
    item_pad = jnp.pad(seq_item, ((0, 0), (0, SEQ_PAD - SEQ)))
    idx, val = _compute_scatter_args(
        all_memory, last_memory, item_pad, Wr, Ur, Vr_w)
    zeros = _make_zeros()
    out_ref = jax.new_ref(zeros.reshape(BATCH * NSEG, 16))
    _make_scatter_kernel()(idx.reshape(-1), val.reshape(-1), out_ref)
    return out_ref[...].reshape(BATCH, NITEM)


# speedup vs baseline: 1.0020x; 1.0020x over previous
import functools

import jax
import jax.numpy as jnp
from jax import lax
from jax.experimental import pallas as pl
from jax.experimental.pallas import tpu as pltpu
from jax.experimental.pallas import tpu_sc as plsc
from jax._src.pallas import mpmd as pl_mpmd

BATCH = 1024
SEQ = 50
HID = 64
NITEM = 100000
SEQ_PAD = 64
NWORK = 32
ROWS_PER_W = BATCH // NWORK
BB = 256
NSEG = NITEM // 16
RING = 4


def _memset_body(o_ref):
    o_ref[...] = jnp.zeros_like(o_ref)


def _make_zeros():
    return pl.pallas_call(
        _memset_body,
        grid=(NWORK,),
        out_specs=pl.BlockSpec((BATCH // NWORK, NITEM), lambda i: (i, 0)),
        out_shape=jax.ShapeDtypeStruct((BATCH, NITEM), jnp.float32),
    )()


def _probs_body(am_ref, lm_ref, item_ref, wr_ref, ur_ref, vrw_ref,
                idx_out, val_out):
    am = am_ref[...]
    lm = lm_ref[...]
    item = item_ref[...]
    wr = wr_ref[...]
    ur = ur_ref[...]
    vrw = vrw_ref[...]

    amu = lax.dot_general(am, ur, (((2,), (1,)), ((), ())),
                          preferred_element_type=jnp.float32)
    lmw = lax.dot_general(lm, wr, (((1,), (1,)), ((), ())),
                          preferred_element_type=jnp.float32)
    t = jnp.tanh(amu + lmw[:, None, :])
    s = jnp.sum(t * vrw[0][None, None, :], axis=-1)
    s = s - jnp.max(s, axis=-1, keepdims=True)
    e = jnp.exp(s)
    p = e / jnp.sum(e, axis=-1, keepdims=True)

    it = item[:, :SEQ]
    eq = it[:, :, None] == it[:, None, :]
    comb = jnp.sum(jnp.where(eq, p[:, None, :], 0.0), axis=-1)
    qlt = (jnp.arange(SEQ)[:, None] > jnp.arange(SEQ)[None, :])[None]
    firsti = jnp.where(
        jnp.sum(jnp.where(eq & qlt, 1, 0), axis=-1) == 0, 1, 0)

    lane = (jnp.arange(SEQ_PAD, dtype=jnp.int32) % 16)[None, :]
    pad_cols = SEQ_PAD - SEQ
    first_p = jnp.pad(firsti, ((0, 0), (0, pad_cols))) > 0
    comb_p = jnp.pad(comb, ((0, 0), (0, pad_cols)))
    it_p = jnp.pad(it, ((0, 0), (0, pad_cols)))
    idx_out[...] = jnp.where(first_p, it_p, NITEM + lane).astype(jnp.int32)
    val_out[...] = jnp.where(first_p, comb_p, 0.0)


def _compute_scatter_args(all_memory, last_memory, seq_item, Wr, Ur, Vr_w):
    grid = BATCH // BB
    return pl.pallas_call(
        _probs_body,
        grid=(grid,),
        in_specs=[
            pl.BlockSpec((BB, SEQ, HID), lambda i: (i, 0, 0)),
            pl.BlockSpec((BB, HID), lambda i: (i, 0)),
            pl.BlockSpec((BB, SEQ_PAD), lambda i: (i, 0)),
            pl.BlockSpec((HID, HID), lambda i: (0, 0)),
            pl.BlockSpec((HID, HID), lambda i: (0, 0)),
            pl.BlockSpec((1, HID), lambda i: (0, 0)),
        ],
        out_specs=[
            pl.BlockSpec((BB, SEQ_PAD), lambda i: (i, 0)),
            pl.BlockSpec((BB, SEQ_PAD), lambda i: (i, 0)),
        ],
        out_shape=[
            jax.ShapeDtypeStruct((BATCH, SEQ_PAD), jnp.int32),
            jax.ShapeDtypeStruct((BATCH, SEQ_PAD), jnp.float32),
        ],
    )(all_memory, last_memory, seq_item, Wr, Ur, Vr_w)


@functools.cache
def _make_scatter_kernel():
    mesh = plsc.VectorSubcoreMesh(core_axis_name="c", subcore_axis_name="s",
                                  num_cores=2, num_subcores=16)
    return pl_mpmd._mpmd_map(
        [(mesh, _scatter_body)],
        out_types=jax.ShapeDtypeStruct((BATCH * NSEG, 16), jnp.float32),
        input_output_aliases={2: 0},
        compiler_params=pltpu.CompilerParams(needs_layout_passes=False,
                                             use_tc_tiling_on_sc=False),
        scratch_types=[
            pltpu.VMEM((NITEM + 16,), jnp.float32),
            pltpu.VMEM((ROWS_PER_W * SEQ_PAD,), jnp.int32),
            pltpu.VMEM((ROWS_PER_W * SEQ_PAD,), jnp.float32),
            [pltpu.VMEM((SEQ_PAD,), jnp.int32) for _ in range(RING)],
            [pltpu.VMEM((SEQ_PAD, 16), jnp.float32) for _ in range(RING)],
            pltpu.SemaphoreType.DMA,
        ],
    )


def _scatter_body(idx_hbm, val_hbm, zeros_in, out3, row_buf, idx_v, val_v,
                  sidx, sdat, sem):
    del zeros_in
    wid = lax.axis_index("s") * 2 + lax.axis_index("c")
    base = wid * ROWS_PER_W

    pltpu.sync_copy(idx_hbm.at[pl.ds(base * SEQ_PAD, ROWS_PER_W * SEQ_PAD)],
                    idx_v)
    pltpu.sync_copy(val_hbm.at[pl.ds(base * SEQ_PAD, ROWS_PER_W * SEQ_PAD)],
                    val_v)

    zeros16 = jnp.zeros((16,), jnp.float32)

    def zinit(i, carry):
        row_buf[pl.ds(i * 16, 16)] = zeros16
        return carry

    lax.fori_loop(0, (NITEM + 16) // 16, zinit, 0)

    def one_row(r, slot):
        off = r * SEQ_PAD
        for k in range(SEQ_PAD // 16):
            idx = idx_v[pl.ds(off + k * 16, 16)]
            val = val_v[pl.ds(off + k * 16, 16)]
            plsc.addupdate_scatter(row_buf, [idx], val)
        row_gseg = (base + r) * NSEG
        vec0 = idx_v[pl.ds(off, 16)]
        seg0 = (vec0[0] >> 4) << 4
        for k in range(SEQ_PAD // 16):
            vec = idx_v[pl.ds(off + k * 16, 16)]
            startv = jnp.where(vec < NITEM, (vec >> 4) << 4, seg0)
            sidx[slot][pl.ds(k * 16, 16)] = row_gseg + (startv >> 4)
            for jj in range(16):
                sdat[slot][k * 16 + jj, :] = row_buf[pl.ds(startv[jj], 16)]
        pltpu.async_copy(sdat[slot], out3.at[sidx[slot]], sem)
        for k in range(SEQ_PAD // 16):
            idx = idx_v[pl.ds(off + k * 16, 16)]
            plsc.store_scatter(row_buf, [idx], zeros16)

    for s in range(RING):
        one_row(s, s)

    def group(g, carry):
        for s in range(RING):
            pltpu.make_async_copy(sdat[s], out3.at[sidx[s]], sem).wait()
            one_row(g * RING + s, s)
        return carry

    lax.fori_loop(1, ROWS_PER_W // RING, group, 0)

    for s in range(RING):
        pltpu.make_async_copy(sdat[s], out3.at[sidx[s]], sem).wait()


def kernel(all_memory, last_memory, seq_item, Wr, Ur, Vr_w, Vr_b):
    del Vr_b
    seq_item = seq_item.astype(jnp.int32)
    item_pad = jnp.pad(seq_item, ((0, 0), (0, SEQ_PAD - SEQ)))
    idx, val = _compute_scatter_args(
        all_memory, last_memory, item_pad, Wr, Ur, Vr_w)
    zeros = _make_zeros()
    out3 = _make_scatter_kernel()(idx.reshape(-1), val.reshape(-1),
                                  zeros.reshape(BATCH * NSEG, 16))
    return out3.reshape(BATCH, NITEM)


# speedup vs baseline: 1.4947x; 1.4916x over previous
import functools

import jax
import jax.numpy as jnp
from jax import lax
from jax.experimental import pallas as pl
from jax.experimental.pallas import tpu as pltpu
from jax.experimental.pallas import tpu_sc as plsc
from jax._src.pallas import mpmd as pl_mpmd

BATCH = 1024
SEQ = 50
HID = 64
NITEM = 100000
SEQ_PAD = 64
NWORK = 32
ROWS_PER_W = BATCH // NWORK
BB = 256
NSEG = NITEM // 16
RING = 4


def _memset_body(o_ref):
    o_ref[...] = jnp.zeros_like(o_ref)


def _make_zeros():
    rows = BATCH * NITEM // 128
    return pl.pallas_call(
        _memset_body,
        grid=(NWORK,),
        out_specs=pl.BlockSpec((rows // NWORK, 128), lambda i: (i, 0)),
        out_shape=jax.ShapeDtypeStruct((rows, 128), jnp.float32),
    )()


def _probs_body(am_ref, lm_ref, item_ref, wr_ref, ur_ref, vrw_ref,
                idx_out, val_out):
    am = am_ref[...]
    lm = lm_ref[...]
    item = item_ref[...]
    wr = wr_ref[...]
    ur = ur_ref[...]
    vrw = vrw_ref[...]

    amu = lax.dot_general(am, ur, (((2,), (1,)), ((), ())),
                          preferred_element_type=jnp.float32)
    lmw = lax.dot_general(lm, wr, (((1,), (1,)), ((), ())),
                          preferred_element_type=jnp.float32)
    t = jnp.tanh(amu + lmw[:, None, :])
    s = jnp.sum(t * vrw[0][None, None, :], axis=-1)
    s = s - jnp.max(s, axis=-1, keepdims=True)
    e = jnp.exp(s)
    p = e / jnp.sum(e, axis=-1, keepdims=True)

    it = item[:, :SEQ]
    eq = it[:, :, None] == it[:, None, :]
    comb = jnp.sum(jnp.where(eq, p[:, None, :], 0.0), axis=-1)
    qlt = (jnp.arange(SEQ)[:, None] > jnp.arange(SEQ)[None, :])[None]
    firsti = jnp.where(
        jnp.sum(jnp.where(eq & qlt, 1, 0), axis=-1) == 0, 1, 0)

    lane = (jnp.arange(SEQ_PAD, dtype=jnp.int32) % 16)[None, :]
    pad_cols = SEQ_PAD - SEQ
    first_p = jnp.pad(firsti, ((0, 0), (0, pad_cols))) > 0
    comb_p = jnp.pad(comb, ((0, 0), (0, pad_cols)))
    it_p = jnp.pad(it, ((0, 0), (0, pad_cols)))
    idx_out[...] = jnp.where(first_p, it_p, NITEM + lane).astype(jnp.int32)
    val_out[...] = jnp.where(first_p, comb_p, 0.0)


def _compute_scatter_args(all_memory, last_memory, seq_item, Wr, Ur, Vr_w):
    grid = BATCH // BB
    return pl.pallas_call(
        _probs_body,
        grid=(grid,),
        in_specs=[
            pl.BlockSpec((BB, SEQ, HID), lambda i: (i, 0, 0)),
            pl.BlockSpec((BB, HID), lambda i: (i, 0)),
            pl.BlockSpec((BB, SEQ_PAD), lambda i: (i, 0)),
            pl.BlockSpec((HID, HID), lambda i: (0, 0)),
            pl.BlockSpec((HID, HID), lambda i: (0, 0)),
            pl.BlockSpec((1, HID), lambda i: (0, 0)),
        ],
        out_specs=[
            pl.BlockSpec((BB, SEQ_PAD), lambda i: (i, 0)),
            pl.BlockSpec((BB, SEQ_PAD), lambda i: (i, 0)),
        ],
        out_shape=[
            jax.ShapeDtypeStruct((BATCH, SEQ_PAD), jnp.int32),
            jax.ShapeDtypeStruct((BATCH, SEQ_PAD), jnp.float32),
        ],
    )(all_memory, last_memory, seq_item, Wr, Ur, Vr_w)


@functools.cache
def _make_scatter_kernel():
    mesh = plsc.VectorSubcoreMesh(core_axis_name="c", subcore_axis_name="s",
                                  num_cores=2, num_subcores=16)
    return pl_mpmd._mpmd_map(
        [(mesh, _scatter_body)],
        out_types=jax.ShapeDtypeStruct((BATCH * NSEG, 16), jnp.float32),
        input_output_aliases={2: 0},
        compiler_params=pltpu.CompilerParams(needs_layout_passes=False,
                                             use_tc_tiling_on_sc=False),
        scratch_types=[
            pltpu.VMEM((NITEM + 16,), jnp.float32),
            pltpu.VMEM((ROWS_PER_W * SEQ_PAD,), jnp.int32),
            pltpu.VMEM((ROWS_PER_W * SEQ_PAD,), jnp.float32),
            [pltpu.VMEM((SEQ_PAD,), jnp.int32) for _ in range(RING)],
            [pltpu.VMEM((SEQ_PAD, 16), jnp.float32) for _ in range(RING)],
            pltpu.SemaphoreType.DMA,
        ],
    )


def _scatter_body(idx_hbm, val_hbm, zeros_in, out3, row_buf, idx_v, val_v,
                  sidx, sdat, sem):
    del zeros_in
    wid = lax.axis_index("s") * 2 + lax.axis_index("c")
    base = wid * ROWS_PER_W

    pltpu.sync_copy(idx_hbm.at[pl.ds(base * SEQ_PAD, ROWS_PER_W * SEQ_PAD)],
                    idx_v)
    pltpu.sync_copy(val_hbm.at[pl.ds(base * SEQ_PAD, ROWS_PER_W * SEQ_PAD)],
                    val_v)

    zeros16 = jnp.zeros((16,), jnp.float32)

    def zinit(i, carry):
        row_buf[pl.ds(i * 16, 16)] = zeros16
        return carry

    lax.fori_loop(0, (NITEM + 16) // 16, zinit, 0)

    def one_row(r, slot):
        off = r * SEQ_PAD
        for k in range(SEQ_PAD // 16):
            idx = idx_v[pl.ds(off + k * 16, 16)]
            val = val_v[pl.ds(off + k * 16, 16)]
            plsc.addupdate_scatter(row_buf, [idx], val)
        row_gseg = (base + r) * NSEG
        vec0 = idx_v[pl.ds(off, 16)]
        seg0 = (vec0[0] >> 4) << 4
        for k in range(SEQ_PAD // 16):
            vec = idx_v[pl.ds(off + k * 16, 16)]
            startv = jnp.where(vec < NITEM, (vec >> 4) << 4, seg0)
            sidx[slot][pl.ds(k * 16, 16)] = row_gseg + (startv >> 4)
            for jj in range(16):
                sdat[slot][k * 16 + jj, :] = row_buf[pl.ds(startv[jj], 16)]
        pltpu.async_copy(sdat[slot], out3.at[sidx[slot]], sem)
        for k in range(SEQ_PAD // 16):
            idx = idx_v[pl.ds(off + k * 16, 16)]
            plsc.store_scatter(row_buf, [idx], zeros16)

    for s in range(RING):
        one_row(s, s)

    def group(g, carry):
        for s in range(RING):
            pltpu.make_async_copy(sdat[s], out3.at[sidx[s]], sem).wait()
            one_row(g * RING + s, s)
        return carry

    lax.fori_loop(1, ROWS_PER_W // RING, group, 0)

    for s in range(RING):
        pltpu.make_async_copy(sdat[s], out3.at[sidx[s]], sem).wait()


def kernel(all_memory, last_memory, seq_item, Wr, Ur, Vr_w, Vr_b):
    del Vr_b
    seq_item = seq_item.astype(jnp.int32)
    item_pad = jnp.pad(seq_item, ((0, 0), (0, SEQ_PAD - SEQ)))
    idx, val = _compute_scatter_args(
        all_memory, last_memory, item_pad, Wr, Ur, Vr_w)
    zeros = _make_zeros()
    out3 = _make_scatter_kernel()(idx.reshape(-1), val.reshape(-1),
                                  zeros.reshape(BATCH * NSEG, 16))
    return out3.reshape(BATCH, NITEM)
